# Initial kernel scaffold; baseline (speedup 1.0000x reference)
import jax
import jax.numpy as jnp
from jax.experimental import pallas as pl


def kernel(nodes, senders, receivers, params):
    raise NotImplementedError("write your pallas kernel here")


# speedup vs baseline: 1.1604x; 1.1604x over previous
import functools

import jax
import jax.numpy as jnp
from jax import lax
from jax.experimental import pallas as pl
from jax.experimental.pallas import tpu as pltpu
from jax.experimental.pallas import tpu_sc as plsc

_N = 10000
_E = 160000
_H = 64
_NC = 2
_NS = 16
_NW = _NC * _NS


def _silu(x):
    return x * jax.nn.sigmoid(x)


def _sc_gather(table, idx, chunk):
    _, w = table.shape
    b = idx.shape[0]
    per_w = b // _NW
    n_chunks = per_w // chunk
    mesh = plsc.VectorSubcoreMesh(core_axis_name="c", subcore_axis_name="s")

    @functools.partial(
        pl.kernel,
        mesh=mesh,
        out_type=jax.ShapeDtypeStruct((b, w), jnp.float32),
        compiler_params=pltpu.CompilerParams(use_tc_tiling_on_sc=False),
        scratch_types=[
            pltpu.VMEM((chunk,), jnp.int32),
            pltpu.VMEM((chunk, w), jnp.float32),
            pltpu.SemaphoreType.DMA,
        ],
    )
    def k(table_hbm, idx_hbm, out_hbm, idx_v, rows_v, sem):
        wid = lax.axis_index("s") * _NC + lax.axis_index("c")
        base = wid * per_w

        @pl.loop(0, n_chunks)
        def _(i):
            off = base + i * chunk
            pltpu.sync_copy(idx_hbm.at[pl.ds(off, chunk)], idx_v)
            pltpu.async_copy(table_hbm.at[idx_v], rows_v, sem).wait()
            pltpu.sync_copy(rows_v, out_hbm.at[pl.ds(off, chunk)])

    return k(table, idx)


def _sc_scatter_add(vals, idx, n_rows, chunk):
    e, w = vals.shape
    per_w = e // _NW
    n_chunks = per_w // chunk
    rps = n_rows // _NS
    mesh = plsc.VectorSubcoreMesh(core_axis_name="c", subcore_axis_name="s")

    @functools.partial(
        pl.kernel,
        mesh=mesh,
        out_type=jax.ShapeDtypeStruct((_NC, n_rows, w), jnp.float32),
        compiler_params=pltpu.CompilerParams(use_tc_tiling_on_sc=False),
        scratch_types=[
            pltpu.VMEM((chunk,), jnp.int32),
            pltpu.VMEM((chunk, w), jnp.float32),
            pltpu.VMEM_SHARED((n_rows, w), jnp.float32),
            pltpu.SemaphoreType.DMA,
        ],
    )
    def k(val_hbm, idx_hbm, out_hbm, idx_v, vals_v, acc_sh, sem):
        c = lax.axis_index("c")
        s = lax.axis_index("s")
        zvec = jnp.zeros((16,), jnp.float32)

        @pl.loop(0, rps)
        def _(r):
            for j in range(w // 16):
                vals_v[r, pl.ds(j * 16, 16)] = zvec

        pltpu.sync_copy(vals_v.at[pl.ds(0, rps)], acc_sh.at[pl.ds(s * rps, rps)])
        plsc.subcore_barrier()

        base = (s * _NC + c) * per_w

        @pl.loop(0, n_chunks)
        def _(i):
            off = base + i * chunk
            pltpu.sync_copy(idx_hbm.at[pl.ds(off, chunk)], idx_v)
            pltpu.sync_copy(val_hbm.at[pl.ds(off, chunk)], vals_v)
            pltpu.sync_copy(vals_v, acc_sh.at[idx_v], add=True)

        plsc.subcore_barrier()
        pltpu.sync_copy(
            acc_sh.at[pl.ds(s * rps, rps)],
            out_hbm.at[c, pl.ds(s * rps, rps)],
        )

    return k(vals, idx)


def _combine(x, a16, b):
    y = x[:, 0:_H]
    for j in range(1, 4):
        y = y + a16[:, j : j + 1] * x[:, j * _H : (j + 1) * _H]
    return y + b


def _dot(a, b):
    return jnp.dot(a, b, preferred_element_type=jnp.float32,
                   precision=lax.Precision.HIGHEST)


def _edge_mlp(gathered, a16_in, wt, wb, w2, w3, b1, b2, b3, first):
    be = 1600
    nb = _E // be
    dnp = gathered.shape[1]

    def body(*refs):
        if first:
            (hs_ref, hr_ref, wt_ref, wb_ref, w2_ref, w3_ref,
             b1_ref, b2_ref, b3_ref, m_ref, a_ref) = refs
        else:
            (hs_ref, hr_ref, a_in_ref, wt_ref, wb_ref, w2_ref, w3_ref,
             b1_ref, b2_ref, b3_ref, m_ref) = refs
        hs = hs_ref[...]
        hr = hr_ref[...]
        if first:
            r = hs[:, 0:3] - hr[:, 0:3]
            norm = jnp.sqrt(jnp.sum(r * r, axis=1, keepdims=True))
            rn = r / (norm + 1e-9)
            ones = jnp.ones((be, 1), jnp.float32)
            a16 = jnp.concatenate(
                [ones, jnp.sqrt(jnp.float32(3.0)) * rn, ones,
                 jnp.zeros((be, 11), jnp.float32)], axis=1)
            a_ref[...] = a16
        else:
            a16 = a_in_ref[...]
        x = _dot(hs, wt_ref[...]) + _dot(hr, wb_ref[...])
        m = _silu(_combine(x, a16, b1_ref[...]))
        m = _silu(_combine(_dot(m, w2_ref[...]), a16, b2_ref[...]))
        m = _combine(_dot(m, w3_ref[...]), a16, b3_ref[...])
        m_ref[...] = m

    full = lambda shape: pl.BlockSpec(shape, lambda i: (0, 0))
    in_specs = [
        pl.BlockSpec((be, dnp), lambda i: (i, 0)),
        pl.BlockSpec((be, dnp), lambda i: (i + nb, 0)),
    ]
    ins = [gathered, gathered]
    if not first:
        in_specs.append(pl.BlockSpec((be, 16), lambda i: (i, 0)))
        ins.append(a16_in)
    in_specs += [full(wt.shape), full(wb.shape), full(w2.shape),
                 full(w3.shape), full(b1.shape), full(b2.shape),
                 full(b3.shape)]
    ins += [wt, wb, w2, w3, b1, b2, b3]
    out_shape = [jax.ShapeDtypeStruct((_E, _H), jnp.float32)]
    out_specs = [pl.BlockSpec((be, _H), lambda i: (i, 0))]
    if first:
        out_shape.append(jax.ShapeDtypeStruct((_E, 16), jnp.float32))
        out_specs.append(pl.BlockSpec((be, 16), lambda i: (i, 0)))
    out = pl.pallas_call(
        body, grid=(nb,), in_specs=in_specs,
        out_specs=out_specs, out_shape=out_shape,
    )(*ins)
    return out


def _node_mlp(h, msum, asum, wh1, wm1, wa1, b1, wx2, wm2, wa2, b2,
              wx3, wm3, wa3, b3, wr, br):
    bn = 2000
    nb = _N // bn
    dnp = h.shape[1]
    dop = wh1.shape[1]

    def body(h_ref, ms_ref, as_ref, wh1_ref, wm1_ref, wa1_ref, b1_ref,
             wx2_ref, wm2_ref, wa2_ref, b2_ref, wx3_ref, wm3_ref, wa3_ref,
             b3_ref, wr_ref, br_ref, out_ref):
        hh = h_ref[...]
        ms = ms_ref[0] + ms_ref[1]
        am = as_ref[0] + as_ref[1]
        cnt = am[:, 4:5]
        inv = 1.0 / jnp.maximum(cnt, 1.0)
        mi = ms * inv
        ai = am * inv
        x1 = _silu(_dot(hh, wh1_ref[...]) + _dot(mi, wm1_ref[...])
                   + _dot(ai, wa1_ref[...]) + b1_ref[...])
        x2 = _silu(_dot(x1, wx2_ref[...]) + _dot(mi, wm2_ref[...])
                   + _dot(ai, wa2_ref[...]) + b2_ref[...])
        x3 = (_dot(x2, wx3_ref[...]) + _dot(mi, wm3_ref[...])
              + _dot(ai, wa3_ref[...]) + b3_ref[...])
        out_ref[...] = x3 + _dot(hh, wr_ref[...]) + br_ref[...]

    full = lambda shape: pl.BlockSpec(shape, lambda i: tuple(0 for _ in shape))
    in_specs = [
        pl.BlockSpec((bn, dnp), lambda i: (i, 0)),
        pl.BlockSpec((2, bn, _H), lambda i: (0, i, 0)),
        pl.BlockSpec((2, bn, 16), lambda i: (0, i, 0)),
    ]
    ws = [wh1, wm1, wa1, b1, wx2, wm2, wa2, b2, wx3, wm3, wa3, b3, wr, br]
    in_specs += [full(w.shape) for w in ws]
    return pl.pallas_call(
        body, grid=(nb,),
        in_specs=in_specs,
        out_specs=pl.BlockSpec((bn, dop), lambda i: (i, 0)),
        out_shape=jax.ShapeDtypeStruct((_N, dop), jnp.float32),
    )(h, msum, asum, *ws)


def _pad_to(x, rows, cols):
    return jnp.pad(x, ((0, rows - x.shape[0]), (0, cols - x.shape[1])))


def _prep_step(p, dn, dnp, do, dop):
    h4 = 4 * _H
    we1 = p['We1']
    wt = _pad_to(we1[:dn].reshape(dn, h4), dnp, h4)
    wb = _pad_to(we1[dn:].reshape(dn, h4), dnp, h4)
    w2 = p['We2'].reshape(_H, h4)
    w3 = p['We3'].reshape(_H, h4)
    b1 = p['be1'].reshape(1, _H)
    b2 = p['be2'].reshape(1, _H)
    b3 = p['be3'].reshape(1, _H)
    wn1, wn2, wn3 = p['Wn1'], p['Wn2'], p['Wn3']
    wh1 = _pad_to(wn1[:dn], dnp, dop)
    wm1 = _pad_to(wn1[dn:dn + _H], _H, dop)
    wa1 = _pad_to(wn1[dn + _H:], 16, dop)
    wx2 = _pad_to(wn2[:do], dop, dop)
    wm2 = _pad_to(wn2[do:do + _H], _H, dop)
    wa2 = _pad_to(wn2[do + _H:], 16, dop)
    wx3 = _pad_to(wn3[:do], dop, dop)
    wm3 = _pad_to(wn3[do:do + _H], _H, dop)
    wa3 = _pad_to(wn3[do + _H:], 16, dop)
    bn1 = _pad_to(p['bn1'].reshape(1, do), 1, dop)
    bn2 = _pad_to(p['bn2'].reshape(1, do), 1, dop)
    bn3 = _pad_to(p['bn3'].reshape(1, do), 1, dop)
    wr = _pad_to(p['Wr'], dnp, dop)
    br = _pad_to(p['br'].reshape(1, do), 1, dop)
    return dict(wt=wt, wb=wb, w2=w2, w3=w3, b1=b1, b2=b2, b3=b3,
                wh1=wh1, wm1=wm1, wa1=wa1, bn1=bn1,
                wx2=wx2, wm2=wm2, wa2=wa2, bn2=bn2,
                wx3=wx3, wm3=wm3, wa3=wa3, bn3=bn3, wr=wr, br=br)


_D_NODES = [131, 64, 64]
_D_NODES_P = [144, 64, 64]
_D_OUTS = [64, 64, 131]
_D_OUTS_P = [64, 64, 144]


def kernel(nodes, senders, receivers, params):
    idx2 = jnp.concatenate([senders, receivers]).astype(jnp.int32)
    recv = receivers.astype(jnp.int32)
    h = jnp.pad(nodes, ((0, 0), (0, _D_NODES_P[0] - _D_NODES[0])))
    a16 = None
    asum = None
    for s in range(3):
        st = _prep_step(params['steps'][s], _D_NODES[s], _D_NODES_P[s],
                        _D_OUTS[s], _D_OUTS_P[s])
        gath = _sc_gather(h, idx2, chunk=200 if s == 0 else 1000)
        if s == 0:
            m3, a16 = _edge_mlp(gath, None, st['wt'], st['wb'], st['w2'],
                                st['w3'], st['b1'], st['b2'], st['b3'],
                                first=True)
            asum = _sc_scatter_add(a16, recv, _N, chunk=1000)
        else:
            (m3,) = _edge_mlp(gath, a16, st['wt'], st['wb'], st['w2'],
                              st['w3'], st['b1'], st['b2'], st['b3'],
                              first=False)
        msum = _sc_scatter_add(m3, recv, _N, chunk=1000)
        h = _node_mlp(h, msum, asum, st['wh1'], st['wm1'], st['wa1'],
                      st['bn1'], st['wx2'], st['wm2'], st['wa2'], st['bn2'],
                      st['wx3'], st['wm3'], st['wa3'], st['bn3'],
                      st['wr'], st['br'])
    return h[:, :_D_OUTS[2]]


# speedup vs baseline: 2.5353x; 2.1847x over previous
import functools

import jax
import jax.numpy as jnp
from jax import lax
from jax.experimental import pallas as pl
from jax.experimental.pallas import tpu as pltpu
from jax.experimental.pallas import tpu_sc as plsc

_N = 10000
_E = 160000
_H = 64
_NC = 2
_NS = 16
_NW = _NC * _NS


def _silu(x):
    return x * jax.nn.sigmoid(x)


def _sc_gather(table, idx, chunk):
    _, w = table.shape
    b = idx.shape[0]
    per_w = b // _NW
    n_chunks = per_w // chunk
    mesh = plsc.VectorSubcoreMesh(core_axis_name="c", subcore_axis_name="s")

    @functools.partial(
        pl.kernel,
        mesh=mesh,
        out_type=jax.ShapeDtypeStruct((b, w), jnp.float32),
        compiler_params=pltpu.CompilerParams(use_tc_tiling_on_sc=False),
        scratch_types=[
            pltpu.VMEM((chunk,), jnp.int32),
            pltpu.VMEM((chunk, w), jnp.float32),
            pltpu.SemaphoreType.DMA,
        ],
    )
    def k(table_hbm, idx_hbm, out_hbm, idx_v, rows_v, sem):
        wid = lax.axis_index("s") * _NC + lax.axis_index("c")
        base = wid * per_w

        @pl.loop(0, n_chunks)
        def _(i):
            off = base + i * chunk
            pltpu.sync_copy(idx_hbm.at[pl.ds(off, chunk)], idx_v)
            pltpu.async_copy(table_hbm.at[idx_v], rows_v, sem).wait()
            pltpu.sync_copy(rows_v, out_hbm.at[pl.ds(off, chunk)])

    return k(table, idx)


def _sc_scatter_add(vals, idx, n_rows, chunk):
    e, w = vals.shape
    per_w = e // _NW
    n_chunks = per_w // chunk
    rps = n_rows // _NS
    mesh = plsc.VectorSubcoreMesh(core_axis_name="c", subcore_axis_name="s")

    @functools.partial(
        pl.kernel,
        mesh=mesh,
        out_type=jax.ShapeDtypeStruct((_NC, n_rows, w), jnp.float32),
        compiler_params=pltpu.CompilerParams(use_tc_tiling_on_sc=False),
        scratch_types=[
            pltpu.VMEM((chunk,), jnp.int32),
            pltpu.VMEM((chunk, w), jnp.float32),
            pltpu.VMEM_SHARED((n_rows, w), jnp.float32),
            pltpu.SemaphoreType.DMA,
        ],
    )
    def k(val_hbm, idx_hbm, out_hbm, idx_v, vals_v, acc_sh, sem):
        c = lax.axis_index("c")
        s = lax.axis_index("s")
        zvec = jnp.zeros((16,), jnp.float32)

        @pl.loop(0, rps)
        def _(r):
            for j in range(w // 16):
                vals_v[r, pl.ds(j * 16, 16)] = zvec

        pltpu.sync_copy(vals_v.at[pl.ds(0, rps)], acc_sh.at[pl.ds(s * rps, rps)])
        plsc.subcore_barrier()

        base = (s * _NC + c) * per_w

        @pl.loop(0, n_chunks)
        def _(i):
            off = base + i * chunk
            pltpu.sync_copy(idx_hbm.at[pl.ds(off, chunk)], idx_v)
            pltpu.sync_copy(val_hbm.at[pl.ds(off, chunk)], vals_v)
            pltpu.sync_copy(vals_v, acc_sh.at[idx_v], add=True)

        plsc.subcore_barrier()
        pltpu.sync_copy(
            acc_sh.at[pl.ds(s * rps, rps)],
            out_hbm.at[c, pl.ds(s * rps, rps)],
        )

    return k(vals, idx)


def _combine(x, a16, b):
    y = x[:, 0:_H]
    for j in range(1, 4):
        y = y + a16[:, j : j + 1] * x[:, j * _H : (j + 1) * _H]
    return y + b


def _dot(a, b):
    return jnp.dot(a, b, preferred_element_type=jnp.float32,
                   precision=lax.Precision.DEFAULT)


def _edge_mlp(gathered, a16_in, wt, wb, w2, w3, b1, b2, b3, first):
    be = 1600
    nb = _E // be
    dnp = gathered.shape[1]

    def body(*refs):
        if first:
            (hs_ref, hr_ref, wt_ref, wb_ref, w2_ref, w3_ref,
             b1_ref, b2_ref, b3_ref, m_ref, a_ref) = refs
        else:
            (hs_ref, hr_ref, a_in_ref, wt_ref, wb_ref, w2_ref, w3_ref,
             b1_ref, b2_ref, b3_ref, m_ref) = refs
        hs = hs_ref[...]
        hr = hr_ref[...]
        if first:
            r = hs[:, 0:3] - hr[:, 0:3]
            norm = jnp.sqrt(jnp.sum(r * r, axis=1, keepdims=True))
            rn = r / (norm + 1e-9)
            ones = jnp.ones((be, 1), jnp.float32)
            a16 = jnp.concatenate(
                [ones, jnp.sqrt(jnp.float32(3.0)) * rn, ones,
                 jnp.zeros((be, 11), jnp.float32)], axis=1)
            a_ref[...] = a16
        else:
            a16 = a_in_ref[...]
        x = _dot(hs, wt_ref[...]) + _dot(hr, wb_ref[...])
        m = _silu(_combine(x, a16, b1_ref[...]))
        m = _silu(_combine(_dot(m, w2_ref[...]), a16, b2_ref[...]))
        m = _combine(_dot(m, w3_ref[...]), a16, b3_ref[...])
        m_ref[...] = m

    full = lambda shape: pl.BlockSpec(shape, lambda i: (0, 0))
    in_specs = [
        pl.BlockSpec((be, dnp), lambda i: (i, 0)),
        pl.BlockSpec((be, dnp), lambda i: (i + nb, 0)),
    ]
    ins = [gathered, gathered]
    if not first:
        in_specs.append(pl.BlockSpec((be, 16), lambda i: (i, 0)))
        ins.append(a16_in)
    in_specs += [full(wt.shape), full(wb.shape), full(w2.shape),
                 full(w3.shape), full(b1.shape), full(b2.shape),
                 full(b3.shape)]
    ins += [wt, wb, w2, w3, b1, b2, b3]
    out_shape = [jax.ShapeDtypeStruct((_E, _H), jnp.float32)]
    out_specs = [pl.BlockSpec((be, _H), lambda i: (i, 0))]
    if first:
        out_shape.append(jax.ShapeDtypeStruct((_E, 16), jnp.float32))
        out_specs.append(pl.BlockSpec((be, 16), lambda i: (i, 0)))
    out = pl.pallas_call(
        body, grid=(nb,), in_specs=in_specs,
        out_specs=out_specs, out_shape=out_shape,
    )(*ins)
    return out


def _node_mlp(h, msum, asum, wh1, wm1, wa1, b1, wx2, wm2, wa2, b2,
              wx3, wm3, wa3, b3, wr, br):
    bn = 2000
    nb = _N // bn
    dnp = h.shape[1]
    dop = wh1.shape[1]

    def body(h_ref, ms_ref, as_ref, wh1_ref, wm1_ref, wa1_ref, b1_ref,
             wx2_ref, wm2_ref, wa2_ref, b2_ref, wx3_ref, wm3_ref, wa3_ref,
             b3_ref, wr_ref, br_ref, out_ref):
        hh = h_ref[...]
        ms = ms_ref[0] + ms_ref[1]
        am = as_ref[0] + as_ref[1]
        cnt = am[:, 4:5]
        inv = 1.0 / jnp.maximum(cnt, 1.0)
        mi = ms * inv
        ai = am * inv
        x1 = _silu(_dot(hh, wh1_ref[...]) + _dot(mi, wm1_ref[...])
                   + _dot(ai, wa1_ref[...]) + b1_ref[...])
        x2 = _silu(_dot(x1, wx2_ref[...]) + _dot(mi, wm2_ref[...])
                   + _dot(ai, wa2_ref[...]) + b2_ref[...])
        x3 = (_dot(x2, wx3_ref[...]) + _dot(mi, wm3_ref[...])
              + _dot(ai, wa3_ref[...]) + b3_ref[...])
        out_ref[...] = x3 + _dot(hh, wr_ref[...]) + br_ref[...]

    full = lambda shape: pl.BlockSpec(shape, lambda i: tuple(0 for _ in shape))
    in_specs = [
        pl.BlockSpec((bn, dnp), lambda i: (i, 0)),
        pl.BlockSpec((2, bn, _H), lambda i: (0, i, 0)),
        pl.BlockSpec((2, bn, 16), lambda i: (0, i, 0)),
    ]
    ws = [wh1, wm1, wa1, b1, wx2, wm2, wa2, b2, wx3, wm3, wa3, b3, wr, br]
    in_specs += [full(w.shape) for w in ws]
    return pl.pallas_call(
        body, grid=(nb,),
        in_specs=in_specs,
        out_specs=pl.BlockSpec((bn, dop), lambda i: (i, 0)),
        out_shape=jax.ShapeDtypeStruct((_N, dop), jnp.float32),
    )(h, msum, asum, *ws)


def _pad_to(x, rows, cols):
    return jnp.pad(x, ((0, rows - x.shape[0]), (0, cols - x.shape[1])))


def _prep_step(p, dn, dnp, do, dop):
    h4 = 4 * _H
    we1 = p['We1']
    wt = _pad_to(we1[:dn].reshape(dn, h4), dnp, h4)
    wb = _pad_to(we1[dn:].reshape(dn, h4), dnp, h4)
    w2 = p['We2'].reshape(_H, h4)
    w3 = p['We3'].reshape(_H, h4)
    b1 = p['be1'].reshape(1, _H)
    b2 = p['be2'].reshape(1, _H)
    b3 = p['be3'].reshape(1, _H)
    wn1, wn2, wn3 = p['Wn1'], p['Wn2'], p['Wn3']
    wh1 = _pad_to(wn1[:dn], dnp, dop)
    wm1 = _pad_to(wn1[dn:dn + _H], _H, dop)
    wa1 = _pad_to(wn1[dn + _H:], 16, dop)
    wx2 = _pad_to(wn2[:do], dop, dop)
    wm2 = _pad_to(wn2[do:do + _H], _H, dop)
    wa2 = _pad_to(wn2[do + _H:], 16, dop)
    wx3 = _pad_to(wn3[:do], dop, dop)
    wm3 = _pad_to(wn3[do:do + _H], _H, dop)
    wa3 = _pad_to(wn3[do + _H:], 16, dop)
    bn1 = _pad_to(p['bn1'].reshape(1, do), 1, dop)
    bn2 = _pad_to(p['bn2'].reshape(1, do), 1, dop)
    bn3 = _pad_to(p['bn3'].reshape(1, do), 1, dop)
    wr = _pad_to(p['Wr'], dnp, dop)
    br = _pad_to(p['br'].reshape(1, do), 1, dop)
    return dict(wt=wt, wb=wb, w2=w2, w3=w3, b1=b1, b2=b2, b3=b3,
                wh1=wh1, wm1=wm1, wa1=wa1, bn1=bn1,
                wx2=wx2, wm2=wm2, wa2=wa2, bn2=bn2,
                wx3=wx3, wm3=wm3, wa3=wa3, bn3=bn3, wr=wr, br=br)


_D_NODES = [131, 64, 64]
_D_NODES_P = [144, 64, 64]
_D_OUTS = [64, 64, 131]
_D_OUTS_P = [64, 64, 144]


def kernel(nodes, senders, receivers, params):
    idx2 = jnp.concatenate([senders, receivers]).astype(jnp.int32)
    recv = receivers.astype(jnp.int32)
    h = jnp.pad(nodes, ((0, 0), (0, _D_NODES_P[0] - _D_NODES[0])))
    a16 = None
    asum = None
    for s in range(3):
        st = _prep_step(params['steps'][s], _D_NODES[s], _D_NODES_P[s],
                        _D_OUTS[s], _D_OUTS_P[s])
        gath = _sc_gather(h, idx2, chunk=200 if s == 0 else 1000)
        if s == 0:
            m3, a16 = _edge_mlp(gath, None, st['wt'], st['wb'], st['w2'],
                                st['w3'], st['b1'], st['b2'], st['b3'],
                                first=True)
            asum = _sc_scatter_add(a16, recv, _N, chunk=1000)
        else:
            (m3,) = _edge_mlp(gath, a16, st['wt'], st['wb'], st['w2'],
                              st['w3'], st['b1'], st['b2'], st['b3'],
                              first=False)
        msum = _sc_scatter_add(m3, recv, _N, chunk=1000)
        h = _node_mlp(h, msum, asum, st['wh1'], st['wm1'], st['wa1'],
                      st['bn1'], st['wx2'], st['wm2'], st['wa2'], st['bn2'],
                      st['wx3'], st['wm3'], st['wa3'], st['bn3'],
                      st['wr'], st['br'])
    return h[:, :_D_OUTS[2]]


# speedup vs baseline: 2.9830x; 1.1766x over previous
import functools

import jax
import jax.numpy as jnp
from jax import lax
from jax.experimental import pallas as pl
from jax.experimental.pallas import tpu as pltpu
from jax.experimental.pallas import tpu_sc as plsc

_N = 10000
_E = 160000
_H = 64
_NC = 2
_NS = 16
_NW = _NC * _NS


def _silu(x):
    return x * jax.nn.sigmoid(x)


def _sc_gather(table, idx, chunk):
    _, w = table.shape
    b = idx.shape[0]
    per_w = b // _NW
    n_chunks = per_w // chunk
    mesh = plsc.VectorSubcoreMesh(core_axis_name="c", subcore_axis_name="s")

    @functools.partial(
        pl.kernel,
        mesh=mesh,
        out_type=jax.ShapeDtypeStruct((b, w), jnp.float32),
        compiler_params=pltpu.CompilerParams(use_tc_tiling_on_sc=True),
        scratch_types=[
            pltpu.VMEM((chunk,), jnp.int32),
            pltpu.VMEM((chunk, w), jnp.float32),
            pltpu.SemaphoreType.DMA,
        ],
    )
    def k(table_hbm, idx_hbm, out_hbm, idx_v, rows_v, sem):
        wid = lax.axis_index("s") * _NC + lax.axis_index("c")
        base = wid * per_w

        @pl.loop(0, n_chunks)
        def _(i):
            off = base + i * chunk
            pltpu.sync_copy(idx_hbm.at[pl.ds(off, chunk)], idx_v)
            pltpu.async_copy(table_hbm.at[idx_v], rows_v, sem).wait()
            pltpu.sync_copy(rows_v, out_hbm.at[pl.ds(off, chunk)])

    return k(table, idx)


def _sc_scatter_add(vals, idx, n_rows, chunk):
    e, w = vals.shape
    per_w = e // _NW
    n_chunks = per_w // chunk
    zrows = 1000
    nz = n_rows // zrows
    zreps = zrows // chunk
    mesh = plsc.VectorSubcoreMesh(core_axis_name="c", subcore_axis_name="s")

    @functools.partial(
        pl.kernel,
        mesh=mesh,
        out_type=jax.ShapeDtypeStruct((_NC, n_rows, w), jnp.float32),
        compiler_params=pltpu.CompilerParams(use_tc_tiling_on_sc=True),
        scratch_types=[
            pltpu.VMEM((chunk,), jnp.int32),
            pltpu.VMEM((chunk, w), jnp.float32),
            pltpu.VMEM_SHARED((n_rows, w), jnp.float32),
            pltpu.SemaphoreType.DMA,
        ],
    )
    def k(val_hbm, idx_hbm, out_hbm, idx_v, vals_v, acc_sh, sem):
        c = lax.axis_index("c")
        s = lax.axis_index("s")
        zvec = jnp.zeros((16,), jnp.float32)

        @pl.loop(0, chunk)
        def _(r):
            for j in range(w // 16):
                vals_v[r, pl.ds(j * 16, 16)] = zvec

        @pl.when(s < nz)
        def _():
            @pl.loop(0, zreps)
            def _(r):
                pltpu.sync_copy(
                    vals_v, acc_sh.at[pl.ds(s * zrows + r * chunk, chunk)])

        plsc.subcore_barrier()

        base = (s * _NC + c) * per_w

        @pl.loop(0, n_chunks)
        def _(i):
            off = base + i * chunk
            pltpu.sync_copy(idx_hbm.at[pl.ds(off, chunk)], idx_v)
            pltpu.sync_copy(val_hbm.at[pl.ds(off, chunk)], vals_v)
            pltpu.sync_copy(vals_v, acc_sh.at[idx_v], add=True)

        plsc.subcore_barrier()

        @pl.when(s < nz)
        def _():
            pltpu.sync_copy(
                acc_sh.at[pl.ds(s * zrows, zrows)],
                out_hbm.at[c, pl.ds(s * zrows, zrows)],
            )

    return k(vals, idx)


def _combine(x, a16, b):
    y = x[:, 0:_H]
    for j in range(1, 4):
        y = y + a16[:, j : j + 1] * x[:, j * _H : (j + 1) * _H]
    return y + b


def _dot(a, b):
    return jnp.dot(a, b, preferred_element_type=jnp.float32,
                   precision=lax.Precision.DEFAULT)


def _edge_mlp(gathered, a16_in, wt, wb, w2, w3, b1, b2, b3, first):
    be = 1600
    nb = _E // be
    dnp = gathered.shape[1]

    def body(*refs):
        if first:
            (hs_ref, hr_ref, wt_ref, wb_ref, w2_ref, w3_ref,
             b1_ref, b2_ref, b3_ref, v_ref, a_ref) = refs
        else:
            (hs_ref, hr_ref, a_in_ref, wt_ref, wb_ref, w2_ref, w3_ref,
             b1_ref, b2_ref, b3_ref, v_ref) = refs
        hs = hs_ref[...]
        hr = hr_ref[...]
        if first:
            r = hs[:, 0:3] - hr[:, 0:3]
            norm = jnp.sqrt(jnp.sum(r * r, axis=1, keepdims=True))
            rn = r / (norm + 1e-9)
            ones = jnp.ones((be, 1), jnp.float32)
            a16 = jnp.concatenate(
                [ones, jnp.sqrt(jnp.float32(3.0)) * rn, ones,
                 jnp.zeros((be, 11), jnp.float32)], axis=1)
            a_ref[...] = a16
        else:
            a16 = a_in_ref[...]
        x = _dot(hs, wt_ref[...]) + _dot(hr, wb_ref[...])
        m = _silu(_combine(x, a16, b1_ref[...]))
        m = _silu(_combine(_dot(m, w2_ref[...]), a16, b2_ref[...]))
        m = _combine(_dot(m, w3_ref[...]), a16, b3_ref[...])
        v_ref[...] = jnp.concatenate(
            [m, a16[:, 0:4],
             jnp.zeros((be, 128 - _H - 4), jnp.float32)], axis=1)

    full = lambda shape: pl.BlockSpec(shape, lambda i: (0, 0))
    in_specs = [
        pl.BlockSpec((be, dnp), lambda i: (i, 0)),
        pl.BlockSpec((be, dnp), lambda i: (i + nb, 0)),
    ]
    ins = [gathered, gathered]
    if not first:
        in_specs.append(pl.BlockSpec((be, 16), lambda i: (i, 0)))
        ins.append(a16_in)
    in_specs += [full(wt.shape), full(wb.shape), full(w2.shape),
                 full(w3.shape), full(b1.shape), full(b2.shape),
                 full(b3.shape)]
    ins += [wt, wb, w2, w3, b1, b2, b3]
    out_shape = [jax.ShapeDtypeStruct((_E, 128), jnp.float32)]
    out_specs = [pl.BlockSpec((be, 128), lambda i: (i, 0))]
    if first:
        out_shape.append(jax.ShapeDtypeStruct((_E, 16), jnp.float32))
        out_specs.append(pl.BlockSpec((be, 16), lambda i: (i, 0)))
    return pl.pallas_call(
        body, grid=(nb,), in_specs=in_specs,
        out_specs=out_specs, out_shape=out_shape,
    )(*ins)


def _node_mlp(h, sums, wh1, wm1, wa1, b1, wx2, wm2, wa2, b2,
              wx3, wm3, wa3, b3, wr, br):
    bn = 2000
    nb = _N // bn
    dnp = h.shape[1]
    dop = wh1.shape[1]

    def body(h_ref, sm_ref, wh1_ref, wm1_ref, wa1_ref, b1_ref,
             wx2_ref, wm2_ref, wa2_ref, b2_ref, wx3_ref, wm3_ref, wa3_ref,
             b3_ref, wr_ref, br_ref, out_ref):
        hh = h_ref[...]
        sm = sm_ref[0] + sm_ref[1]
        cnt = sm[:, _H : _H + 1]
        inv = 1.0 / jnp.maximum(cnt, 1.0)
        mi = sm[:, 0:_H] * inv
        ai = sm[:, _H : _H + 16] * inv
        x1 = _silu(_dot(hh, wh1_ref[...]) + _dot(mi, wm1_ref[...])
                   + _dot(ai, wa1_ref[...]) + b1_ref[...])
        x2 = _silu(_dot(x1, wx2_ref[...]) + _dot(mi, wm2_ref[...])
                   + _dot(ai, wa2_ref[...]) + b2_ref[...])
        x3 = (_dot(x2, wx3_ref[...]) + _dot(mi, wm3_ref[...])
              + _dot(ai, wa3_ref[...]) + b3_ref[...])
        out_ref[...] = x3 + _dot(hh, wr_ref[...]) + br_ref[...]

    full = lambda shape: pl.BlockSpec(shape, lambda i: tuple(0 for _ in shape))
    in_specs = [
        pl.BlockSpec((bn, dnp), lambda i: (i, 0)),
        pl.BlockSpec((2, bn, 128), lambda i: (0, i, 0)),
    ]
    ws = [wh1, wm1, wa1, b1, wx2, wm2, wa2, b2, wx3, wm3, wa3, b3, wr, br]
    in_specs += [full(w.shape) for w in ws]
    return pl.pallas_call(
        body, grid=(nb,),
        in_specs=in_specs,
        out_specs=pl.BlockSpec((bn, dop), lambda i: (i, 0)),
        out_shape=jax.ShapeDtypeStruct((_N, dop), jnp.float32),
    )(h, sums, *ws)


def _pad_to(x, rows, cols):
    return jnp.pad(x, ((0, rows - x.shape[0]), (0, cols - x.shape[1])))


def _prep_step(p, dn, dnp, do, dop):
    h4 = 4 * _H
    we1 = p['We1']
    wt = _pad_to(we1[:dn].reshape(dn, h4), dnp, h4)
    wb = _pad_to(we1[dn:].reshape(dn, h4), dnp, h4)
    w2 = p['We2'].reshape(_H, h4)
    w3 = p['We3'].reshape(_H, h4)
    b1 = p['be1'].reshape(1, _H)
    b2 = p['be2'].reshape(1, _H)
    b3 = p['be3'].reshape(1, _H)
    wn1, wn2, wn3 = p['Wn1'], p['Wn2'], p['Wn3']
    wh1 = _pad_to(wn1[:dn], dnp, dop)
    wm1 = _pad_to(wn1[dn:dn + _H], _H, dop)
    wa1 = _pad_to(wn1[dn + _H:], 16, dop)
    wx2 = _pad_to(wn2[:do], dop, dop)
    wm2 = _pad_to(wn2[do:do + _H], _H, dop)
    wa2 = _pad_to(wn2[do + _H:], 16, dop)
    wx3 = _pad_to(wn3[:do], dop, dop)
    wm3 = _pad_to(wn3[do:do + _H], _H, dop)
    wa3 = _pad_to(wn3[do + _H:], 16, dop)
    bn1 = _pad_to(p['bn1'].reshape(1, do), 1, dop)
    bn2 = _pad_to(p['bn2'].reshape(1, do), 1, dop)
    bn3 = _pad_to(p['bn3'].reshape(1, do), 1, dop)
    wr = _pad_to(p['Wr'], dnp, dop)
    br = _pad_to(p['br'].reshape(1, do), 1, dop)
    return dict(wt=wt, wb=wb, w2=w2, w3=w3, b1=b1, b2=b2, b3=b3,
                wh1=wh1, wm1=wm1, wa1=wa1, bn1=bn1,
                wx2=wx2, wm2=wm2, wa2=wa2, bn2=bn2,
                wx3=wx3, wm3=wm3, wa3=wa3, bn3=bn3, wr=wr, br=br)


_D_NODES = [131, 64, 64]
_D_NODES_P = [256, 128, 128]
_D_OUTS = [64, 64, 131]
_D_OUTS_P = [128, 128, 144]


def kernel(nodes, senders, receivers, params):
    idx2 = jnp.concatenate([senders, receivers]).astype(jnp.int32)
    recv = receivers.astype(jnp.int32)
    h = jnp.pad(nodes, ((0, 0), (0, _D_NODES_P[0] - _D_NODES[0])))
    a16 = None
    for s in range(3):
        st = _prep_step(params['steps'][s], _D_NODES[s], _D_NODES_P[s],
                        _D_OUTS[s], _D_OUTS_P[s])
        gath = _sc_gather(h, idx2, chunk=200)
        if s == 0:
            vals, a16 = _edge_mlp(gath, None, st['wt'], st['wb'], st['w2'],
                                  st['w3'], st['b1'], st['b2'], st['b3'],
                                  first=True)
        else:
            (vals,) = _edge_mlp(gath, a16, st['wt'], st['wb'], st['w2'],
                                st['w3'], st['b1'], st['b2'], st['b3'],
                                first=False)
        sums = _sc_scatter_add(vals, recv, _N, chunk=200)
        h = _node_mlp(h, sums, st['wh1'], st['wm1'], st['wa1'],
                      st['bn1'], st['wx2'], st['wm2'], st['wa2'], st['bn2'],
                      st['wx3'], st['wm3'], st['wa3'], st['bn3'],
                      st['wr'], st['br'])
    return h[:, :_D_OUTS[2]]


# speedup vs baseline: 3.3623x; 1.1272x over previous
import functools

import jax
import jax.numpy as jnp
from jax import lax
from jax.experimental import pallas as pl
from jax.experimental.pallas import tpu as pltpu
from jax.experimental.pallas import tpu_sc as plsc

_N = 10000
_E = 160000
_H = 64
_NC = 2
_NS = 16
_NW = _NC * _NS


def _silu(x):
    return x * jax.nn.sigmoid(x)


def _sc_gather(table, idx, chunk):
    _, w = table.shape
    b = idx.shape[0]
    per_w = b // _NW
    n_chunks = per_w // chunk
    nh = n_chunks // 2
    mesh = plsc.VectorSubcoreMesh(core_axis_name="c", subcore_axis_name="s")

    @functools.partial(
        pl.kernel,
        mesh=mesh,
        out_type=jax.ShapeDtypeStruct((b, w), jnp.float32),
        compiler_params=pltpu.CompilerParams(use_tc_tiling_on_sc=True),
        scratch_types=[
            pltpu.VMEM((chunk,), jnp.int32),
            pltpu.VMEM((chunk,), jnp.int32),
            pltpu.VMEM((chunk, w), jnp.float32),
            pltpu.VMEM((chunk, w), jnp.float32),
            pltpu.SemaphoreType.DMA,
            pltpu.SemaphoreType.DMA,
            pltpu.SemaphoreType.DMA,
            pltpu.SemaphoreType.DMA,
            pltpu.SemaphoreType.DMA,
            pltpu.SemaphoreType.DMA,
        ],
    )
    def k(table_hbm, idx_hbm, out_hbm, idx0, idx1, rows0, rows1,
          si0, si1, sg0, sg1, sw0, sw1):
        wid = lax.axis_index("s") * _NC + lax.axis_index("c")
        base = wid * per_w
        idx_v = (idx0, idx1)
        rows_v = (rows0, rows1)
        si = (si0, si1)
        sg = (sg0, sg1)
        sw = (sw0, sw1)

        def start_idx(i, sl):
            pltpu.async_copy(idx_hbm.at[pl.ds(base + i * chunk, chunk)],
                             idx_v[sl], si[sl])

        def wait(sem, dst_like_src, dst):
            pltpu.make_async_copy(dst_like_src, dst, sem).wait()

        start_idx(0, 0)
        start_idx(1, 1)

        @pl.loop(0, nh)
        def _(kk):
            for sl in (0, 1):
                i = 2 * kk + sl

                @pl.when(kk > 0)
                def _():
                    wait(sw[sl], rows_v[sl],
                         out_hbm.at[pl.ds(base, chunk)])
                wait(si[sl], idx_hbm.at[pl.ds(base, chunk)], idx_v[sl])
                pltpu.async_copy(table_hbm.at[idx_v[sl]], rows_v[sl], sg[sl])

            for sl in (0, 1):
                i = 2 * kk + sl
                wait(sg[sl], table_hbm.at[pl.ds(0, chunk)], rows_v[sl])
                pltpu.async_copy(
                    rows_v[sl],
                    out_hbm.at[pl.ds(base + i * chunk, chunk)], sw[sl])

            @pl.when(kk < nh - 1)
            def _():
                start_idx(2 * kk + 2, 0)
                start_idx(2 * kk + 3, 1)

        for sl in (0, 1):
            wait(sw[sl], rows_v[sl], out_hbm.at[pl.ds(base, chunk)])

    return k(table, idx)


def _sc_scatter_add(vals, idx, n_rows, chunk):
    e, w = vals.shape
    per_w = e // _NW
    n_chunks = per_w // chunk
    zrows = 1000
    nz = n_rows // zrows
    zreps = zrows // chunk
    mesh = plsc.VectorSubcoreMesh(core_axis_name="c", subcore_axis_name="s")

    @functools.partial(
        pl.kernel,
        mesh=mesh,
        out_type=jax.ShapeDtypeStruct((_NC, n_rows, w), jnp.float32),
        compiler_params=pltpu.CompilerParams(use_tc_tiling_on_sc=True),
        scratch_types=[
            pltpu.VMEM((chunk,), jnp.int32),
            pltpu.VMEM((chunk, w), jnp.float32),
            pltpu.VMEM_SHARED((n_rows, w), jnp.float32),
            pltpu.SemaphoreType.DMA,
        ],
    )
    def k(val_hbm, idx_hbm, out_hbm, idx_v, vals_v, acc_sh, sem):
        c = lax.axis_index("c")
        s = lax.axis_index("s")
        zvec = jnp.zeros((16,), jnp.float32)

        @pl.loop(0, chunk)
        def _(r):
            for j in range(w // 16):
                vals_v[r, pl.ds(j * 16, 16)] = zvec

        @pl.when(s < nz)
        def _():
            @pl.loop(0, zreps)
            def _(r):
                pltpu.sync_copy(
                    vals_v, acc_sh.at[pl.ds(s * zrows + r * chunk, chunk)])

        plsc.subcore_barrier()

        base = (s * _NC + c) * per_w

        @pl.loop(0, n_chunks)
        def _(i):
            off = base + i * chunk
            pltpu.sync_copy(idx_hbm.at[pl.ds(off, chunk)], idx_v)
            pltpu.sync_copy(val_hbm.at[pl.ds(off, chunk)], vals_v)
            pltpu.sync_copy(vals_v, acc_sh.at[idx_v], add=True)

        plsc.subcore_barrier()

        @pl.when(s < nz)
        def _():
            pltpu.sync_copy(
                acc_sh.at[pl.ds(s * zrows, zrows)],
                out_hbm.at[c, pl.ds(s * zrows, zrows)],
            )

    return k(vals, idx)


def _combine(x, a16, b):
    y = x[:, 0:_H]
    for j in range(1, 4):
        y = y + a16[:, j : j + 1] * x[:, j * _H : (j + 1) * _H]
    return y + b


def _dot(a, b):
    return jnp.dot(a, b, preferred_element_type=jnp.float32,
                   precision=lax.Precision.DEFAULT)


def _edge_mlp(gathered, a16_in, wt, wb, w2, w3, b1, b2, b3, first):
    be = 3200
    nb = _E // be
    dnp = gathered.shape[1]

    def body(*refs):
        if first:
            (hs_ref, hr_ref, wt_ref, wb_ref, w2_ref, w3_ref,
             b1_ref, b2_ref, b3_ref, v_ref, a_ref) = refs
        else:
            (hs_ref, hr_ref, a_in_ref, wt_ref, wb_ref, w2_ref, w3_ref,
             b1_ref, b2_ref, b3_ref, v_ref) = refs
        hs = hs_ref[...]
        hr = hr_ref[...]
        if first:
            r = hs[:, 0:3] - hr[:, 0:3]
            norm = jnp.sqrt(jnp.sum(r * r, axis=1, keepdims=True))
            rn = r / (norm + 1e-9)
            ones = jnp.ones((be, 1), jnp.float32)
            a16 = jnp.concatenate(
                [ones, jnp.sqrt(jnp.float32(3.0)) * rn, ones,
                 jnp.zeros((be, 11), jnp.float32)], axis=1)
            a_ref[...] = a16
        else:
            a16 = a_in_ref[...]
        x = _dot(hs, wt_ref[...]) + _dot(hr, wb_ref[...])
        m = _silu(_combine(x, a16, b1_ref[...]))
        m = _silu(_combine(_dot(m, w2_ref[...]), a16, b2_ref[...]))
        m = _combine(_dot(m, w3_ref[...]), a16, b3_ref[...])
        v_ref[...] = jnp.concatenate(
            [m, a16[:, 0:4],
             jnp.zeros((be, 128 - _H - 4), jnp.float32)], axis=1)

    full = lambda shape: pl.BlockSpec(shape, lambda i: (0, 0))
    in_specs = [
        pl.BlockSpec((be, dnp), lambda i: (i, 0)),
        pl.BlockSpec((be, dnp), lambda i: (i + nb, 0)),
    ]
    ins = [gathered, gathered]
    if not first:
        in_specs.append(pl.BlockSpec((be, 16), lambda i: (i, 0)))
        ins.append(a16_in)
    in_specs += [full(wt.shape), full(wb.shape), full(w2.shape),
                 full(w3.shape), full(b1.shape), full(b2.shape),
                 full(b3.shape)]
    ins += [wt, wb, w2, w3, b1, b2, b3]
    out_shape = [jax.ShapeDtypeStruct((_E, 128), jnp.float32)]
    out_specs = [pl.BlockSpec((be, 128), lambda i: (i, 0))]
    if first:
        out_shape.append(jax.ShapeDtypeStruct((_E, 16), jnp.float32))
        out_specs.append(pl.BlockSpec((be, 16), lambda i: (i, 0)))
    return pl.pallas_call(
        body, grid=(nb,), in_specs=in_specs,
        out_specs=out_specs, out_shape=out_shape,
    )(*ins)


def _node_mlp(h, sums, wh1, wm1, wa1, b1, wx2, wm2, wa2, b2,
              wx3, wm3, wa3, b3, wr, br):
    bn = 2000
    nb = _N // bn
    dnp = h.shape[1]
    dop = wh1.shape[1]

    def body(h_ref, sm_ref, wh1_ref, wm1_ref, wa1_ref, b1_ref,
             wx2_ref, wm2_ref, wa2_ref, b2_ref, wx3_ref, wm3_ref, wa3_ref,
             b3_ref, wr_ref, br_ref, out_ref):
        hh = h_ref[...]
        sm = sm_ref[0] + sm_ref[1]
        cnt = sm[:, _H : _H + 1]
        inv = 1.0 / jnp.maximum(cnt, 1.0)
        mi = sm[:, 0:_H] * inv
        ai = sm[:, _H : _H + 16] * inv
        x1 = _silu(_dot(hh, wh1_ref[...]) + _dot(mi, wm1_ref[...])
                   + _dot(ai, wa1_ref[...]) + b1_ref[...])
        x2 = _silu(_dot(x1, wx2_ref[...]) + _dot(mi, wm2_ref[...])
                   + _dot(ai, wa2_ref[...]) + b2_ref[...])
        x3 = (_dot(x2, wx3_ref[...]) + _dot(mi, wm3_ref[...])
              + _dot(ai, wa3_ref[...]) + b3_ref[...])
        out_ref[...] = x3 + _dot(hh, wr_ref[...]) + br_ref[...]

    full = lambda shape: pl.BlockSpec(shape, lambda i: tuple(0 for _ in shape))
    in_specs = [
        pl.BlockSpec((bn, dnp), lambda i: (i, 0)),
        pl.BlockSpec((2, bn, 128), lambda i: (0, i, 0)),
    ]
    ws = [wh1, wm1, wa1, b1, wx2, wm2, wa2, b2, wx3, wm3, wa3, b3, wr, br]
    in_specs += [full(w.shape) for w in ws]
    return pl.pallas_call(
        body, grid=(nb,),
        in_specs=in_specs,
        out_specs=pl.BlockSpec((bn, dop), lambda i: (i, 0)),
        out_shape=jax.ShapeDtypeStruct((_N, dop), jnp.float32),
    )(h, sums, *ws)


def _pad_to(x, rows, cols):
    return jnp.pad(x, ((0, rows - x.shape[0]), (0, cols - x.shape[1])))


def _prep_step(p, dn, dnp, do, dop):
    h4 = 4 * _H
    we1 = p['We1']
    wt = _pad_to(we1[:dn].reshape(dn, h4), dnp, h4)
    wb = _pad_to(we1[dn:].reshape(dn, h4), dnp, h4)
    w2 = p['We2'].reshape(_H, h4)
    w3 = p['We3'].reshape(_H, h4)
    b1 = p['be1'].reshape(1, _H)
    b2 = p['be2'].reshape(1, _H)
    b3 = p['be3'].reshape(1, _H)
    wn1, wn2, wn3 = p['Wn1'], p['Wn2'], p['Wn3']
    wh1 = _pad_to(wn1[:dn], dnp, dop)
    wm1 = _pad_to(wn1[dn:dn + _H], _H, dop)
    wa1 = _pad_to(wn1[dn + _H:], 16, dop)
    wx2 = _pad_to(wn2[:do], dop, dop)
    wm2 = _pad_to(wn2[do:do + _H], _H, dop)
    wa2 = _pad_to(wn2[do + _H:], 16, dop)
    wx3 = _pad_to(wn3[:do], dop, dop)
    wm3 = _pad_to(wn3[do:do + _H], _H, dop)
    wa3 = _pad_to(wn3[do + _H:], 16, dop)
    bn1 = _pad_to(p['bn1'].reshape(1, do), 1, dop)
    bn2 = _pad_to(p['bn2'].reshape(1, do), 1, dop)
    bn3 = _pad_to(p['bn3'].reshape(1, do), 1, dop)
    wr = _pad_to(p['Wr'], dnp, dop)
    br = _pad_to(p['br'].reshape(1, do), 1, dop)
    return dict(wt=wt, wb=wb, w2=w2, w3=w3, b1=b1, b2=b2, b3=b3,
                wh1=wh1, wm1=wm1, wa1=wa1, bn1=bn1,
                wx2=wx2, wm2=wm2, wa2=wa2, bn2=bn2,
                wx3=wx3, wm3=wm3, wa3=wa3, bn3=bn3, wr=wr, br=br)


_D_NODES = [131, 64, 64]
_D_NODES_P = [256, 128, 128]
_D_OUTS = [64, 64, 131]
_D_OUTS_P = [128, 128, 144]


def kernel(nodes, senders, receivers, params):
    idx2 = jnp.concatenate([senders, receivers]).astype(jnp.int32)
    recv = receivers.astype(jnp.int32)
    h = jnp.pad(nodes, ((0, 0), (0, _D_NODES_P[0] - _D_NODES[0])))
    a16 = None
    for s in range(3):
        st = _prep_step(params['steps'][s], _D_NODES[s], _D_NODES_P[s],
                        _D_OUTS[s], _D_OUTS_P[s])
        gath = _sc_gather(h, idx2, chunk=200)
        if s == 0:
            vals, a16 = _edge_mlp(gath, None, st['wt'], st['wb'], st['w2'],
                                  st['w3'], st['b1'], st['b2'], st['b3'],
                                  first=True)
        else:
            (vals,) = _edge_mlp(gath, a16, st['wt'], st['wb'], st['w2'],
                                st['w3'], st['b1'], st['b2'], st['b3'],
                                first=False)
        sums = _sc_scatter_add(vals, recv, _N, chunk=200)
        h = _node_mlp(h, sums, st['wh1'], st['wm1'], st['wa1'],
                      st['bn1'], st['wx2'], st['wm2'], st['wa2'], st['bn2'],
                      st['wx3'], st['wm3'], st['wa3'], st['bn3'],
                      st['wr'], st['br'])
    return h[:, :_D_OUTS[2]]
